# baseline (speedup 1.0000x reference)
import functools

import jax
import jax.numpy as jnp
from jax import lax
from jax.experimental import pallas as pl
from jax.experimental.pallas import tpu as pltpu
from jax.experimental.pallas import tpu_sc as plsc

VOCAB = 50368
HID = 1024
HALF = HID // 2
EPS = 1e-5

NC = 2
NS = 16
NW = NC * NS
LANES = 16

N_TOKENS = 4 * 4096
P = 4
STAGE = N_TOKENS // P
TOK_PER_W = STAGE // NW
CHUNK = 16
N_CHUNKS = TOK_PER_W // CHUNK
NBUF = 4
PREF = 3

TC_BLK = 1024
TC_BLOCKS = STAGE // TC_BLK


def _gather_pack_kernel(ids_hbm, table_hbm, out_hbm, idx_v, rows_v, pk_v,
                        gs0, gs1, gs2, gs3, os0, os1, os2, os3):
    wid = lax.axis_index("s") * NC + lax.axis_index("c")
    base = wid * TOK_PER_W
    gsem = (gs0, gs1, gs2, gs3)
    osem = (os0, os1, os2, os3)

    pltpu.sync_copy(ids_hbm.at[pl.ds(base, TOK_PER_W)], idx_v)

    def gather_copy(c, s):
        return pltpu.make_async_copy(
            table_hbm.at[idx_v.at[pl.ds(c * CHUNK, CHUNK)]],
            rows_v.at[s], gsem[s])

    def out_copy(c, s):
        return pltpu.make_async_copy(
            pk_v.at[s], out_hbm.at[pl.ds(base + c * CHUNK, CHUNK)], osem[s])

    def pack_chunk(s):
        rnd = jnp.uint32(0x8000)
        himask = jnp.uint32(0xFFFF0000)
        ushift = jnp.uint32(16)

        def row_body(t, _):
            for g in range(HALF // LANES):
                a = rows_v[s, t, pl.ds(g * LANES, LANES)]
                b = rows_v[s, t, pl.ds(HALF + g * LANES, LANES)]
                ab = lax.bitcast_convert_type(a, jnp.uint32) + rnd
                bb = lax.bitcast_convert_type(b, jnp.uint32) + rnd
                word = (ab >> ushift) | (bb & himask)
                pk_v[s, t, pl.ds(g * LANES, LANES)] = (
                    lax.bitcast_convert_type(word, jnp.float32))
            return 0

        lax.fori_loop(0, CHUNK, row_body, 0)

    for c in range(min(PREF, N_CHUNKS)):
        gather_copy(c, c % NBUF).start()

    def round_body(i, _):
        for k in range(NBUF):
            def _step(i=i, k=k):
                c = NBUF * i + k
                gather_copy(c, k).wait()

                def _drain():
                    out_copy(c - NBUF, k).wait()

                pl.when(c >= NBUF)(_drain)
                pack_chunk(k)
                out_copy(c, k).start()

                def _refill():
                    gather_copy(c + PREF, (k + PREF) % NBUF).start()

                pl.when(c + PREF < N_CHUNKS)(_refill)

            _step()
        return 0

    assert N_CHUNKS % NBUF == 0, (N_CHUNKS, NBUF)
    lax.fori_loop(0, N_CHUNKS // NBUF, round_body, 0)

    for c in range(max(0, N_CHUNKS - NBUF), N_CHUNKS):
        out_copy(c, c % NBUF).wait()


def _sc_gather_pack(ids_stage, table):
    mesh = plsc.VectorSubcoreMesh(core_axis_name="c", subcore_axis_name="s")
    return pl.kernel(
        _gather_pack_kernel,
        out_type=jax.ShapeDtypeStruct((STAGE, HALF), jnp.float32),
        mesh=mesh,
        scratch_types=[
            pltpu.VMEM((TOK_PER_W,), jnp.int32),
            pltpu.VMEM((NBUF, CHUNK, HID), jnp.float32),
            pltpu.VMEM((NBUF, CHUNK, HALF), jnp.float32),
        ] + [pltpu.SemaphoreType.DMA] * (2 * NBUF),
    )(ids_stage, table)


def _ln_block_packed(w, g):
    u = lax.bitcast_convert_type(w, jnp.uint32)
    hi = lax.bitcast_convert_type(u & jnp.uint32(0xFFFF0000), jnp.float32)
    lo = lax.bitcast_convert_type(u << 16, jnp.float32)
    s = jnp.sum(lo, axis=1, keepdims=True) + jnp.sum(hi, axis=1, keepdims=True)
    mean = s * (1.0 / HID)
    dlo = lo - mean
    dhi = hi - mean
    var = (jnp.sum(dlo * dlo, axis=1, keepdims=True)
           + jnp.sum(dhi * dhi, axis=1, keepdims=True)) * (1.0 / HID)
    scl = lax.rsqrt(var + EPS)
    return dlo * scl * g[:, :HALF], dhi * scl * g[:, HALF:]


def _tc_ln_first(x_ref, g_ref, o_ref):
    out_lo, out_hi = _ln_block_packed(x_ref[...], g_ref[...])
    o_ref[:, :HALF] = out_lo
    o_ref[:, HALF:] = out_hi


def _tc_ln_chained(x_ref, g_ref, buf_ref, o_ref):
    del buf_ref
    out_lo, out_hi = _ln_block_packed(x_ref[...], g_ref[...])
    o_ref[:, :HALF] = out_lo
    o_ref[:, HALF:] = out_hi


def _tc_layer_norm_stage(emb_words, gamma2d, buf, k):
    out_spec = pl.BlockSpec(
        (TC_BLK, HID), lambda i, k=k: (k * TC_BLOCKS + i, 0))
    in_specs = [
        pl.BlockSpec((TC_BLK, HALF), lambda i: (i, 0)),
        pl.BlockSpec((1, HID), lambda i: (0, 0)),
    ]
    args = [emb_words, gamma2d]
    if buf is None:
        body = _tc_ln_first
        aliases = {}
    else:
        body = _tc_ln_chained
        in_specs.append(pl.BlockSpec(memory_space=pl.ANY))
        args.append(buf)
        aliases = {2: 0}
    return pl.pallas_call(
        body,
        grid=(TC_BLOCKS,),
        in_specs=in_specs,
        out_specs=out_spec,
        out_shape=jax.ShapeDtypeStruct((N_TOKENS, HID), jnp.float32),
        input_output_aliases=aliases,
    )(*args)


@jax.jit
def kernel(input_ids, table, gamma):
    ids_flat = input_ids.reshape(-1).astype(jnp.int32)
    gamma2d = gamma.reshape(1, HID)
    embs = [_sc_gather_pack(ids_flat[k * STAGE:(k + 1) * STAGE], table)
            for k in range(P)]
    buf = None
    for k in range(P):
        buf = _tc_layer_norm_stage(embs[k], gamma2d, buf, k)
    return buf.reshape(input_ids.shape + (HID,))


# speedup vs baseline: 1.4698x; 1.4698x over previous
import functools

import jax
import jax.numpy as jnp
from jax import lax
from jax.experimental import pallas as pl
from jax.experimental.pallas import tpu as pltpu
from jax.experimental.pallas import tpu_sc as plsc

VOCAB = 50368
HID = 1024
EPS = 1e-5

NC = 2
NS = 16
NW = NC * NS

N_TOKENS = 4 * 4096
STAGES = (2048, 5120, 6144, 3072)
CHUNK = 16
NBUF = 6
PREF = 4
LAG = 2

TC_BLK = 1024


def _make_gather_kernel(tok_per_w, n_chunks):
    def gather_kernel(ids_hbm, table_hbm, out_hbm, idx_v, rows_v, *sems):
        gsem = sems[:NBUF]
        osem = sems[NBUF:]
        wid = lax.axis_index("s") * NC + lax.axis_index("c")
        base = wid * tok_per_w

        pltpu.sync_copy(ids_hbm.at[pl.ds(base, tok_per_w)], idx_v)

        def gather_copy(c):
            return pltpu.make_async_copy(
                table_hbm.at[idx_v.at[pl.ds(c * CHUNK, CHUNK)]],
                rows_v.at[c % NBUF], gsem[c % NBUF])

        def out_copy(c):
            return pltpu.make_async_copy(
                rows_v.at[c % NBUF],
                out_hbm.at[pl.ds(base + c * CHUNK, CHUNK)],
                osem[c % NBUF])

        for c in range(min(PREF, n_chunks)):
            gather_copy(c).start()
        for c in range(n_chunks):
            gather_copy(c).wait()
            out_copy(c).start()
            if c + PREF < n_chunks:
                if c - LAG >= 0:
                    out_copy(c - LAG).wait()
                gather_copy(c + PREF).start()
        for c in range(max(0, n_chunks - NBUF), n_chunks):
            out_copy(c).wait()

    return gather_kernel


def _sc_gather(ids_stage, table):
    stage = ids_stage.shape[0]
    tok_per_w = stage // NW
    n_chunks = tok_per_w // CHUNK
    mesh = plsc.VectorSubcoreMesh(core_axis_name="c", subcore_axis_name="s")
    return pl.kernel(
        _make_gather_kernel(tok_per_w, n_chunks),
        out_type=jax.ShapeDtypeStruct((stage, HID), jnp.float32),
        mesh=mesh,
        scratch_types=[
            pltpu.VMEM((tok_per_w,), jnp.int32),
            pltpu.VMEM((NBUF, CHUNK, HID), jnp.float32),
        ] + [pltpu.SemaphoreType.DMA] * (2 * NBUF),
    )(ids_stage, table)


def _ln_block(x, g):
    mean = jnp.mean(x, axis=1, keepdims=True)
    xc = x - mean
    var = jnp.mean(xc * xc, axis=1, keepdims=True)
    return xc * lax.rsqrt(var + EPS) * g


def _tc_ln_first(x_ref, g_ref, o_ref):
    o_ref[...] = _ln_block(x_ref[...], g_ref[...])


def _tc_ln_chained(x_ref, g_ref, buf_ref, o_ref):
    del buf_ref
    o_ref[...] = _ln_block(x_ref[...], g_ref[...])


def _tc_layer_norm_stage(emb, gamma2d, buf, blk_off):
    stage = emb.shape[0]
    blocks = stage // TC_BLK
    out_spec = pl.BlockSpec(
        (TC_BLK, HID), lambda i, blk_off=blk_off: (blk_off + i, 0))
    in_specs = [
        pl.BlockSpec((TC_BLK, HID), lambda i: (i, 0)),
        pl.BlockSpec((1, HID), lambda i: (0, 0)),
    ]
    args = [emb, gamma2d]
    if buf is None:
        body = _tc_ln_first
        aliases = {}
    else:
        body = _tc_ln_chained
        in_specs.append(pl.BlockSpec(memory_space=pl.ANY))
        args.append(buf)
        aliases = {2: 0}
    return pl.pallas_call(
        body,
        grid=(blocks,),
        in_specs=in_specs,
        out_specs=out_spec,
        out_shape=jax.ShapeDtypeStruct((N_TOKENS, HID), jnp.float32),
        input_output_aliases=aliases,
    )(*args)


@jax.jit
def kernel(input_ids, table, gamma):
    ids_flat = input_ids.reshape(-1).astype(jnp.int32)
    gamma2d = gamma.reshape(1, HID)
    offs = [0]
    for s in STAGES:
        offs.append(offs[-1] + s)
    embs = [_sc_gather(ids_flat[offs[k]:offs[k + 1]], table)
            for k in range(len(STAGES))]
    buf = None
    for k in range(len(STAGES)):
        buf = _tc_layer_norm_stage(embs[k], gamma2d, buf, offs[k] // TC_BLK)
    return buf.reshape(input_ids.shape + (HID,))


# speedup vs baseline: 1.5201x; 1.0343x over previous
import functools

import jax
import jax.numpy as jnp
from jax import lax
from jax.experimental import pallas as pl
from jax.experimental.pallas import tpu as pltpu
from jax.experimental.pallas import tpu_sc as plsc

VOCAB = 50368
HID = 1024
EPS = 1e-5

NC = 2
NS = 16
NW = NC * NS
LANES = 16
VREGS_PER_ROW = HID // LANES

N_TOKENS = 4 * 4096
TOK_PER_W = N_TOKENS // NW
CHUNK = 32
N_CHUNKS = TOK_PER_W // CHUNK


def _lane_shuffle(v, perm):
    return lax.gather(
        v, perm[:, None],
        dimension_numbers=lax.GatherDimensionNumbers(
            offset_dims=(), collapsed_slice_dims=(0,), start_index_map=(0,)),
        slice_sizes=(1,),
        mode=lax.GatherScatterMode.PROMISE_IN_BOUNDS,
    )


def _rsqrt_newton(x):
    i = lax.bitcast_convert_type(x, jnp.int32)
    i = 0x5F3759DF - lax.shift_right_arithmetic(i, 1)
    y = lax.bitcast_convert_type(i, jnp.float32)
    for _ in range(3):
        y = y * (1.5 - 0.5 * x * y * y)
    return y


def _tree_sum(xs):
    xs = list(xs)
    while len(xs) > 1:
        nxt = [a + b for a, b in zip(xs[0::2], xs[1::2])]
        if len(xs) % 2:
            nxt.append(xs[-1])
        xs = nxt
    return xs[0]


ROW_BLOCK = 16
JU = 8


NBUF = 3


def _layer_norm_chunk(rows_v, gamma_v, p_v, q_v):
    lane = lax.iota(jnp.int32, LANES)
    perms = [jnp.bitwise_xor(lane, k) for k in (8, 4, 2, 1)]

    def row_stats(t):
        s_parts, s2_parts = [], []
        for g8 in range(VREGS_PER_ROW // 8):
            vs = [rows_v[t, pl.ds((g8 * 8 + k) * LANES, LANES)]
                  for k in range(8)]
            s_parts.append(_tree_sum(vs))
            s2_parts.append(_tree_sum([v * v for v in vs]))
        return _tree_sum(s_parts), _tree_sum(s2_parts)

    def row_body(tp, _):
        stats = [row_stats(tp * 4 + u) for u in range(4)]
        for u, (s, s2) in enumerate(stats):
            t = tp * 4 + u
            for perm in perms:
                s = s + _lane_shuffle(s, perm)
                s2 = s2 + _lane_shuffle(s2, perm)
            mean = s * (1.0 / HID)
            var = s2 * (1.0 / HID) - mean * mean
            scale = _rsqrt_newton(var + EPS)
            p_v[t, pl.ds(0, LANES)] = scale
            q_v[t, pl.ds(0, LANES)] = mean * scale
        return 0

    lax.fori_loop(0, CHUNK // 4, row_body, 0)

    def pb_body(rb, _):
        t0 = rb * ROW_BLOCK
        ps = [p_v[t0 + r, pl.ds(0, LANES)] for r in range(ROW_BLOCK)]
        qs = [q_v[t0 + r, pl.ds(0, LANES)] for r in range(ROW_BLOCK)]

        def j_body(j, _):
            for u in range(JU):
                jj = j * JU + u
                g = gamma_v[pl.ds(jj * LANES, LANES)]
                for r in range(ROW_BLOCK):
                    v = rows_v[t0 + r, pl.ds(jj * LANES, LANES)]
                    rows_v[t0 + r, pl.ds(jj * LANES, LANES)] = (
                        (v * ps[r] - qs[r]) * g)
            return 0

        lax.fori_loop(0, VREGS_PER_ROW // JU, j_body, 0)
        return 0

    lax.fori_loop(0, CHUNK // ROW_BLOCK, pb_body, 0)


def _ln_kernel(ids_hbm, table_hbm, gamma_hbm, out_hbm,
               idx_v, rows_v, gamma_v, p_v, q_v,
               gs0, gs1, gs2, os0, os1, os2):
    wid = lax.axis_index("s") * NC + lax.axis_index("c")
    base = wid * TOK_PER_W
    gsem = (gs0, gs1, gs2)
    osem = (os0, os1, os2)

    pltpu.sync_copy(gamma_hbm, gamma_v)
    pltpu.sync_copy(ids_hbm.at[pl.ds(base, TOK_PER_W)], idx_v)

    def gather_copy(c, s):
        return pltpu.make_async_copy(
            table_hbm.at[idx_v.at[pl.ds(c * CHUNK, CHUNK)]],
            rows_v.at[s], gsem[s])

    def out_copy(c, s):
        return pltpu.make_async_copy(
            rows_v.at[s], out_hbm.at[pl.ds(base + c * CHUNK, CHUNK)], osem[s])

    def chunk_step(c, s, pred):
        gather_copy(c, s).wait()
        sp = (s + 2) % NBUF

        def _prefetch():
            out_copy(c - 1, sp).wait()
            gather_copy(c + 2, sp).start()

        if pred is None:
            _prefetch()
        else:
            pl.when(pred)(_prefetch)
        _layer_norm_chunk(rows_v.at[s], gamma_v, p_v, q_v)
        out_copy(c, s).start()

    gather_copy(0, 0).start()
    gather_copy(1, 1).start()

    gather_copy(0, 0).wait()
    gather_copy(2, 2).start()
    _layer_norm_chunk(rows_v.at[0], gamma_v, p_v, q_v)
    out_copy(0, 0).start()

    def round_body(i, _):
        c0 = 3 * i
        chunk_step(c0 + 1, 1, None)
        chunk_step(c0 + 2, 2, c0 + 4 < N_CHUNKS)
        chunk_step(c0 + 3, 0, c0 + 5 < N_CHUNKS)
        return 0

    lax.fori_loop(0, (N_CHUNKS - 1) // 3, round_body, 0)

    out_copy(N_CHUNKS - 3, 1).wait()
    out_copy(N_CHUNKS - 2, 2).wait()
    out_copy(N_CHUNKS - 1, 0).wait()


@jax.jit
def kernel(input_ids, table, gamma):
    ids_flat = input_ids.reshape(-1).astype(jnp.int32)
    mesh = plsc.VectorSubcoreMesh(core_axis_name="c", subcore_axis_name="s")
    out = pl.kernel(
        _ln_kernel,
        out_type=jax.ShapeDtypeStruct((N_TOKENS, HID), jnp.float32),
        mesh=mesh,
        scratch_types=[
            pltpu.VMEM((TOK_PER_W,), jnp.int32),
            pltpu.VMEM((NBUF, CHUNK, HID), jnp.float32),
            pltpu.VMEM((HID,), jnp.float32),
            pltpu.VMEM((CHUNK, LANES), jnp.float32),
            pltpu.VMEM((CHUNK, LANES), jnp.float32),
            pltpu.SemaphoreType.DMA,
            pltpu.SemaphoreType.DMA,
            pltpu.SemaphoreType.DMA,
            pltpu.SemaphoreType.DMA,
            pltpu.SemaphoreType.DMA,
            pltpu.SemaphoreType.DMA,
        ],
    )(ids_flat, table, gamma)
    return out.reshape(input_ids.shape + (HID,))
